# baseline (speedup 1.0000x reference)
import functools
import math

import jax
import jax.numpy as jnp
from jax import lax
from jax.experimental import pallas as pl
from jax.experimental.pallas import tpu as pltpu
from jax.experimental.pallas import tpu_sc as plsc

D_MODEL = 64
SCALE = math.sqrt(D_MODEL)

NUM_CORES = 2
NUM_SUBCORES = 16
NW = NUM_CORES * NUM_SUBCORES

CHUNK = 128


def _body(n_chunks, x_hbm, table_hbm, out_hbm, idx_v, rows_v, sem):
  wid = lax.axis_index("s") * NUM_CORES + lax.axis_index("c")
  base = wid * (n_chunks * CHUNK)

  def chunk_body(g, carry):
    off = base + g * CHUNK
    pltpu.sync_copy(x_hbm.at[pl.ds(off, CHUNK)], idx_v)
    pltpu.async_copy(table_hbm.at[idx_v], rows_v, sem).wait()

    def scale_row(i, c):
      for j in range(D_MODEL // 16):
        s = pl.ds(j * 16, 16)
        rows_v[i, s] = rows_v[i, s] * SCALE
      return c

    lax.fori_loop(0, CHUNK, scale_row, 0)
    pltpu.sync_copy(rows_v, out_hbm.at[pl.ds(off, CHUNK)])
    return carry

  lax.fori_loop(0, n_chunks, chunk_body, 0)


def kernel(x, table):
  b0, b1 = x.shape
  n = b0 * b1
  assert n % (NW * CHUNK) == 0
  n_chunks = n // (NW * CHUNK)
  xf = x.reshape(n).astype(jnp.int32)

  mesh = plsc.VectorSubcoreMesh(core_axis_name="c", subcore_axis_name="s")
  gather = functools.partial(
      pl.kernel,
      mesh=mesh,
      out_type=jax.ShapeDtypeStruct((n, D_MODEL), jnp.float32),
      scratch_types=[
          pltpu.VMEM((CHUNK,), jnp.int32),
          pltpu.VMEM((CHUNK, D_MODEL), jnp.float32),
          pltpu.SemaphoreType.DMA,
      ],
      compiler_params=pltpu.CompilerParams(use_tc_tiling_on_sc=False),
  )(functools.partial(_body, n_chunks))

  out = gather(xf, table)
  return out.reshape(b0, b1, D_MODEL)


# speedup vs baseline: 1.2714x; 1.2714x over previous
import functools
import math

import jax
import jax.numpy as jnp
from jax import lax
from jax.experimental import pallas as pl
from jax.experimental.pallas import tpu as pltpu
from jax.experimental.pallas import tpu_sc as plsc

D_MODEL = 64
SCALE = math.sqrt(D_MODEL)

NUM_CORES = 2
NUM_SUBCORES = 16
NW = NUM_CORES * NUM_SUBCORES

IDX_W = 128
GATHERS = 4
CHUNK = IDX_W * GATHERS


def _body(n_chunks, x_hbm, table_hbm, out_hbm, idx_all, rows0, rows1,
          sg0, sg1, so0, so1):
  wid = lax.axis_index("s") * NUM_CORES + lax.axis_index("c")
  idx_rows = n_chunks * GATHERS
  base = wid * (n_chunks * CHUNK)

  pltpu.sync_copy(x_hbm.at[pl.ds(wid * idx_rows, idx_rows)], idx_all)

  def fire_gather(c, rows, sem):
    for j in range(GATHERS):
      pltpu.async_copy(table_hbm.at[idx_all.at[c * GATHERS + j]],
                       rows.at[pl.ds(j * IDX_W, IDX_W)], sem)

  def drain_gather(c, rows, sem):
    for j in range(GATHERS):
      pltpu.make_async_copy(table_hbm.at[idx_all.at[c * GATHERS + j]],
                            rows.at[pl.ds(j * IDX_W, IDX_W)], sem).wait()

  def scale(rows):
    @plsc.parallel_loop(0, CHUNK, unroll=8)
    def _(i):
      for j in range(D_MODEL // 16):
        s = pl.ds(j * 16, 16)
        rows[i, s] = rows[i, s] * SCALE

  def start_out(c, rows, sem):
    pltpu.async_copy(rows, out_hbm.at[pl.ds(base + c * CHUNK, CHUNK)], sem)

  def drain_out(c, rows, sem):
    pltpu.make_async_copy(rows, out_hbm.at[pl.ds(base + c * CHUNK, CHUNK)],
                          sem).wait()

  fire_gather(0, rows0, sg0)

  def pair(gg, carry):
    c0 = 2 * gg
    c1 = c0 + 1
    drain_gather(c0, rows0, sg0)

    @pl.when(gg > 0)
    def _():
      drain_out(c1 - 2, rows1, so1)

    fire_gather(c1, rows1, sg1)
    scale(rows0)
    start_out(c0, rows0, so0)
    drain_gather(c1, rows1, sg1)
    drain_out(c0, rows0, so0)

    @pl.when(c1 + 1 < n_chunks)
    def _():
      fire_gather(c1 + 1, rows0, sg0)

    scale(rows1)
    start_out(c1, rows1, so1)
    return carry

  lax.fori_loop(0, n_chunks // 2, pair, 0)
  drain_out(n_chunks - 1, rows1, so1)


def kernel(x, table):
  b0, b1 = x.shape
  n = b0 * b1
  assert n % (NW * CHUNK) == 0
  n_chunks = n // (NW * CHUNK)
  assert n_chunks % 2 == 0
  xf = x.reshape(n // IDX_W, IDX_W).astype(jnp.int32)

  mesh = plsc.VectorSubcoreMesh(core_axis_name="c", subcore_axis_name="s")
  gather = functools.partial(
      pl.kernel,
      mesh=mesh,
      out_type=jax.ShapeDtypeStruct((n, D_MODEL), jnp.float32),
      scratch_types=[
          pltpu.VMEM((n_chunks * GATHERS, IDX_W), jnp.int32),
          pltpu.VMEM((CHUNK, D_MODEL), jnp.float32),
          pltpu.VMEM((CHUNK, D_MODEL), jnp.float32),
          pltpu.SemaphoreType.DMA,
          pltpu.SemaphoreType.DMA,
          pltpu.SemaphoreType.DMA,
          pltpu.SemaphoreType.DMA,
      ],
      compiler_params=pltpu.CompilerParams(use_tc_tiling_on_sc=False),
  )(functools.partial(_body, n_chunks))

  out = gather(xf, table)
  return out.reshape(b0, b1, D_MODEL)
